# baseline (speedup 1.0000x reference)
import functools

import jax
import jax.numpy as jnp
from jax import lax
from jax.experimental import pallas as pl
from jax.experimental.pallas import tpu as pltpu
from jax.experimental.pallas import tpu_sc as plsc

NC = 2
NS = 16
NW = NC * NS

RPC = 4
GROUPS = ((0, 0), (0, 16), (0, 32), (0, 34),)


def _sc_body(ids_ref, base_ref, prior_ref, gate_ref, out_ref,
             idx2_v, idx_v, base_a, base_b, prior_a, prior_b, gate_a, gate_b,
             out_v, sem_a, sem_b, *, rows_per_worker, l, d):
    wid = lax.axis_index("s") * NC + lax.axis_index("c")
    chunk = RPC * l
    row0 = wid * rows_per_worker
    id0 = row0 * l
    n_chunks = rows_per_worker // RPC
    n_pairs = n_chunks // 2

    pltpu.sync_copy(ids_ref.at[pl.ds(row0, rows_per_worker)], idx2_v)

    def compact(r, _):
        for lo in (0, 16, 32, 34):
            idx_v[pl.ds(r * l + lo, 16)] = idx2_v[r, pl.ds(lo, 16)]
        return 0

    lax.fori_loop(0, rows_per_worker, compact, 0)

    dnums = lax.GatherDimensionNumbers(
        offset_dims=(), collapsed_slice_dims=(0,), start_index_map=(0,))

    SEGS = [(0, 128), (128, 72)]

    def fire(c, base_v, prior_v, gate_v, sem):
        for off, ln in SEGS:
            idx = idx_v.at[pl.ds(c * chunk + off, ln)]
            pltpu.async_copy(base_ref.at[idx], base_v.at[pl.ds(off, ln)], sem)
            pltpu.async_copy(prior_ref.at[idx], prior_v.at[pl.ds(off, ln)], sem)
            pltpu.async_copy(gate_ref.at[idx], gate_v.at[pl.ds(off, ln)], sem)

    def wait(base_v, prior_v, gate_v, sem):
        for off, ln in SEGS:
            pltpu.make_async_copy(
                base_ref.at[pl.ds(0, ln)], base_v.at[pl.ds(off, ln)], sem).wait()
            pltpu.make_async_copy(
                prior_ref.at[pl.ds(0, ln)], prior_v.at[pl.ds(off, ln)], sem).wait()
            pltpu.make_async_copy(
                gate_ref.at[pl.ds(0, ln)], gate_v.at[pl.ds(off, ln)], sem).wait()

    def combine(base_v, prior_v, gate_v):
        def q_body(q, _):
            r0 = q * l
            for lo, js in ((0, range(16)), (16, range(16)), (32, range(16)),
                           (34, (14, 15))):
                g16 = gate_v[pl.ds(r0 + lo, 16)]
                w16 = 1.0 / (1.0 + jnp.exp(-g16))
                for j in js:
                    row = r0 + lo + j
                    w = lax.gather(
                        w16, jnp.full((16, 1), j, jnp.int32), dnums,
                        slice_sizes=(1,),
                        mode=lax.GatherScatterMode.PROMISE_IN_BOUNDS)
                    for k in range(d // 16):
                        sl = pl.ds(k * 16, 16)
                        out_v[q, lo + j, sl] = (
                            base_v[row, sl] + w * prior_v[row, sl])
            return 0

        lax.fori_loop(0, RPC, q_body, 0)

    def writeback(c):
        off = row0 + c * RPC
        pltpu.sync_copy(out_v, out_ref.at[pl.ds(off, RPC), pl.ds(0, l), pl.ds(0, d)])

    fire(0, base_a, prior_a, gate_a, sem_a)

    def pair_body(t, _):
        ca = 2 * t
        wait(base_a, prior_a, gate_a, sem_a)
        fire(ca + 1, base_b, prior_b, gate_b, sem_b)
        combine(base_a, prior_a, gate_a)
        writeback(ca)
        wait(base_b, prior_b, gate_b, sem_b)

        @pl.when(t < n_pairs - 1)
        def _():
            fire(ca + 2, base_a, prior_a, gate_a, sem_a)

        combine(base_b, prior_b, gate_b)
        writeback(ca + 1)
        return 0

    lax.fori_loop(0, n_pairs, pair_body, 0)


def kernel(input_ids, base_weight, prior_matrix, gate_logits):
    b, l = input_ids.shape
    v, d = base_weight.shape
    n = b * l
    assert b % (NW * 2 * RPC) == 0 and d % 16 == 0 and l == 50
    rows_per_worker = b // NW

    ids_p = jnp.pad(input_ids, ((0, 0), (0, 128 - l)))

    mesh = plsc.VectorSubcoreMesh(core_axis_name="c", subcore_axis_name="s")
    body = functools.partial(_sc_body, rows_per_worker=rows_per_worker, l=l, d=d)
    chunk = RPC * l
    call = pl.kernel(
        body,
        mesh=mesh,
        compiler_params=pltpu.CompilerParams(use_tc_tiling_on_sc=False),
        out_type=jax.ShapeDtypeStruct((b, 56, 128), jnp.float32),
        scratch_types=[
            pltpu.VMEM((rows_per_worker, 128), jnp.int32),
            pltpu.VMEM((rows_per_worker * l,), jnp.int32),
            pltpu.VMEM((chunk, d), jnp.float32),
            pltpu.VMEM((chunk, d), jnp.float32),
            pltpu.VMEM((chunk, d), jnp.float32),
            pltpu.VMEM((chunk, d), jnp.float32),
            pltpu.VMEM((chunk,), jnp.float32),
            pltpu.VMEM((chunk,), jnp.float32),
            pltpu.VMEM((RPC, l, d), jnp.float32),
            pltpu.SemaphoreType.DMA,
            pltpu.SemaphoreType.DMA,
        ],
    )
    out = call(ids_p, base_weight, prior_matrix, gate_logits)
    return out[:, :l, :d]


# speedup vs baseline: 1.0071x; 1.0071x over previous
import functools

import jax
import jax.numpy as jnp
from jax import lax
from jax.experimental import pallas as pl
from jax.experimental.pallas import tpu as pltpu
from jax.experimental.pallas import tpu_sc as plsc

NC = 2
NS = 16
NW = NC * NS

RPC = 4
GROUPS = ((0, 0), (0, 16), (0, 32), (0, 34),)


def _sc_body(ids_ref, base_ref, prior_ref, gate_ref, out_ref,
             idx_v, base_a, base_b, prior_a, prior_b, gate_a, gate_b,
             out_v, sem_a, sem_b, *, rows_per_worker, l, d):
    wid = lax.axis_index("s") * NC + lax.axis_index("c")
    chunk = RPC * l
    row0 = wid * rows_per_worker
    id0 = row0 * l
    n_chunks = rows_per_worker // RPC
    n_pairs = n_chunks // 2

    pltpu.sync_copy(ids_ref.at[pl.ds(id0, rows_per_worker * l)], idx_v)

    dnums = lax.GatherDimensionNumbers(
        offset_dims=(), collapsed_slice_dims=(0,), start_index_map=(0,))

    SEGS = [(0, 128), (128, 72)]

    def fire(c, base_v, prior_v, gate_v, sem):
        for off, ln in SEGS:
            idx = idx_v.at[pl.ds(c * chunk + off, ln)]
            pltpu.async_copy(base_ref.at[idx], base_v.at[pl.ds(off, ln)], sem)
            pltpu.async_copy(prior_ref.at[idx], prior_v.at[pl.ds(off, ln)], sem)
            pltpu.async_copy(gate_ref.at[idx], gate_v.at[pl.ds(off, ln)], sem)

    def wait(base_v, prior_v, gate_v, sem):
        for off, ln in SEGS:
            pltpu.make_async_copy(
                base_ref.at[pl.ds(0, ln)], base_v.at[pl.ds(off, ln)], sem).wait()
            pltpu.make_async_copy(
                prior_ref.at[pl.ds(0, ln)], prior_v.at[pl.ds(off, ln)], sem).wait()
            pltpu.make_async_copy(
                gate_ref.at[pl.ds(0, ln)], gate_v.at[pl.ds(off, ln)], sem).wait()

    def combine(base_v, prior_v, gate_v):
        def q_body(q, _):
            r0 = q * l
            for lo, js in ((0, range(16)), (16, range(16)), (32, range(16)),
                           (34, (14, 15))):
                g16 = gate_v[pl.ds(r0 + lo, 16)]
                w16 = 1.0 / (1.0 + jnp.exp(-g16))
                for j in js:
                    row = r0 + lo + j
                    w = lax.gather(
                        w16, jnp.full((16, 1), j, jnp.int32), dnums,
                        slice_sizes=(1,),
                        mode=lax.GatherScatterMode.PROMISE_IN_BOUNDS)
                    for k in range(d // 16):
                        sl = pl.ds(k * 16, 16)
                        out_v[q, lo + j, sl] = (
                            base_v[row, sl] + w * prior_v[row, sl])
            return 0

        lax.fori_loop(0, RPC, q_body, 0)

    def writeback(c):
        off = row0 + c * RPC
        pltpu.sync_copy(out_v, out_ref.at[pl.ds(off, RPC), pl.ds(0, l), pl.ds(0, d)])

    fire(0, base_a, prior_a, gate_a, sem_a)

    def pair_body(t, _):
        ca = 2 * t
        wait(base_a, prior_a, gate_a, sem_a)
        fire(ca + 1, base_b, prior_b, gate_b, sem_b)
        combine(base_a, prior_a, gate_a)
        writeback(ca)
        wait(base_b, prior_b, gate_b, sem_b)

        @pl.when(t < n_pairs - 1)
        def _():
            fire(ca + 2, base_a, prior_a, gate_a, sem_a)

        combine(base_b, prior_b, gate_b)
        writeback(ca + 1)
        return 0

    lax.fori_loop(0, n_pairs, pair_body, 0)


def kernel(input_ids, base_weight, prior_matrix, gate_logits):
    b, l = input_ids.shape
    v, d = base_weight.shape
    n = b * l
    assert b % (NW * 2 * RPC) == 0 and d % 16 == 0 and l == 50
    rows_per_worker = b // NW

    ids1 = input_ids.reshape(n)

    mesh = plsc.VectorSubcoreMesh(core_axis_name="c", subcore_axis_name="s")
    body = functools.partial(_sc_body, rows_per_worker=rows_per_worker, l=l, d=d)
    chunk = RPC * l
    call = pl.kernel(
        body,
        mesh=mesh,
        compiler_params=pltpu.CompilerParams(use_tc_tiling_on_sc=False),
        out_type=jax.ShapeDtypeStruct((b, 56, 128), jnp.float32),
        scratch_types=[
            pltpu.VMEM((rows_per_worker * l,), jnp.int32),
            pltpu.VMEM((chunk, d), jnp.float32),
            pltpu.VMEM((chunk, d), jnp.float32),
            pltpu.VMEM((chunk, d), jnp.float32),
            pltpu.VMEM((chunk, d), jnp.float32),
            pltpu.VMEM((chunk,), jnp.float32),
            pltpu.VMEM((chunk,), jnp.float32),
            pltpu.VMEM((RPC, l, d), jnp.float32),
            pltpu.SemaphoreType.DMA,
            pltpu.SemaphoreType.DMA,
        ],
    )
    out = call(ids1, base_weight, prior_matrix, gate_logits)
    return out[:, :l, :d]


# speedup vs baseline: 1.0432x; 1.0358x over previous
import functools

import jax
import jax.numpy as jnp
from jax import lax
from jax.experimental import pallas as pl
from jax.experimental.pallas import tpu as pltpu
from jax.experimental.pallas import tpu_sc as plsc

NC = 2
NS = 16
NW = NC * NS

RPC = 4
GROUPS = ((0, 0), (0, 16), (0, 32), (0, 34),)


def _sc_body(ids_ref, base_ref, prior_ref, gate_ref, out_ref,
             idx_v, base_a, base_b, base_c, base_d,
             prior_a, prior_b, prior_c, prior_d,
             gate_a, gate_b, gate_c, gate_d,
             out_v, sem_a, sem_b, sem_c, sem_d, *, rows_per_worker, l, d):
    wid = lax.axis_index("s") * NC + lax.axis_index("c")
    chunk = RPC * l
    row0 = wid * rows_per_worker
    id0 = row0 * l
    n_chunks = rows_per_worker // RPC
    n_pairs = n_chunks // 2

    pltpu.sync_copy(ids_ref.at[pl.ds(id0, rows_per_worker * l)], idx_v)

    dnums = lax.GatherDimensionNumbers(
        offset_dims=(), collapsed_slice_dims=(0,), start_index_map=(0,))

    SEGS = [(0, 128), (128, 72)]

    def fire(c, base_v, prior_v, gate_v, sem):
        for off, ln in SEGS:
            idx = idx_v.at[pl.ds(c * chunk + off, ln)]
            pltpu.async_copy(base_ref.at[idx], base_v.at[pl.ds(off, ln)], sem)
            pltpu.async_copy(prior_ref.at[idx], prior_v.at[pl.ds(off, ln)], sem)
            pltpu.async_copy(gate_ref.at[idx], gate_v.at[pl.ds(off, ln)], sem)

    def wait(base_v, prior_v, gate_v, sem):
        for off, ln in SEGS:
            pltpu.make_async_copy(
                base_ref.at[pl.ds(0, ln)], base_v.at[pl.ds(off, ln)], sem).wait()
            pltpu.make_async_copy(
                prior_ref.at[pl.ds(0, ln)], prior_v.at[pl.ds(off, ln)], sem).wait()
            pltpu.make_async_copy(
                gate_ref.at[pl.ds(0, ln)], gate_v.at[pl.ds(off, ln)], sem).wait()

    def combine(base_v, prior_v, gate_v):
        def q_body(q, _):
            r0 = q * l
            for lo, js in ((0, range(16)), (16, range(16)), (32, range(16)),
                           (34, (14, 15))):
                g16 = gate_v[pl.ds(r0 + lo, 16)]
                w16 = 1.0 / (1.0 + jnp.exp(-g16))
                for j in js:
                    row = r0 + lo + j
                    w = lax.gather(
                        w16, jnp.full((16, 1), j, jnp.int32), dnums,
                        slice_sizes=(1,),
                        mode=lax.GatherScatterMode.PROMISE_IN_BOUNDS)
                    for k in range(d // 16):
                        sl = pl.ds(k * 16, 16)
                        out_v[q, lo + j, sl] = (
                            base_v[row, sl] + w * prior_v[row, sl])
            return 0

        lax.fori_loop(0, RPC, q_body, 0)

    def writeback(c):
        off = row0 + c * RPC
        pltpu.sync_copy(out_v, out_ref.at[pl.ds(off, RPC), pl.ds(0, l), pl.ds(0, d)])

    sets = ((base_a, prior_a, gate_a, sem_a),
            (base_b, prior_b, gate_b, sem_b),
            (base_c, prior_c, gate_c, sem_c),
            (base_d, prior_d, gate_d, sem_d))
    n_quads = n_chunks // 4

    fire(0, *sets[0])
    fire(1, *sets[1])

    def quad_body(t, _):
        c = 4 * t
        for i in range(4):
            wait(*sets[i])
            nxt = c + i + 2
            if i < 2:
                fire(nxt, *sets[(i + 2) % 4])
            else:
                @pl.when(t < n_quads - 1)
                def _():
                    fire(nxt, *sets[(i + 2) % 4])
            combine(*sets[i][:3])
            writeback(c + i)
        return 0

    lax.fori_loop(0, n_quads, quad_body, 0)


def kernel(input_ids, base_weight, prior_matrix, gate_logits):
    b, l = input_ids.shape
    v, d = base_weight.shape
    n = b * l
    assert b % (NW * 2 * RPC) == 0 and d % 16 == 0 and l == 50
    rows_per_worker = b // NW

    ids1 = input_ids.reshape(n)

    mesh = plsc.VectorSubcoreMesh(core_axis_name="c", subcore_axis_name="s")
    body = functools.partial(_sc_body, rows_per_worker=rows_per_worker, l=l, d=d)
    chunk = RPC * l
    call = pl.kernel(
        body,
        mesh=mesh,
        compiler_params=pltpu.CompilerParams(use_tc_tiling_on_sc=False),
        out_type=jax.ShapeDtypeStruct((b, 56, 128), jnp.float32),
        scratch_types=[
            pltpu.VMEM((rows_per_worker * l,), jnp.int32),
            pltpu.VMEM((chunk, d), jnp.float32),
            pltpu.VMEM((chunk, d), jnp.float32),
            pltpu.VMEM((chunk, d), jnp.float32),
            pltpu.VMEM((chunk, d), jnp.float32),
            pltpu.VMEM((chunk, d), jnp.float32),
            pltpu.VMEM((chunk, d), jnp.float32),
            pltpu.VMEM((chunk, d), jnp.float32),
            pltpu.VMEM((chunk, d), jnp.float32),
            pltpu.VMEM((chunk,), jnp.float32),
            pltpu.VMEM((chunk,), jnp.float32),
            pltpu.VMEM((chunk,), jnp.float32),
            pltpu.VMEM((chunk,), jnp.float32),
            pltpu.VMEM((RPC, l, d), jnp.float32),
            pltpu.SemaphoreType.DMA,
            pltpu.SemaphoreType.DMA,
            pltpu.SemaphoreType.DMA,
            pltpu.SemaphoreType.DMA,
        ],
    )
    out = call(ids1, base_weight, prior_matrix, gate_logits)
    return out[:, :l, :d]
